# Initial kernel scaffold; baseline (speedup 1.0000x reference)
import jax
import jax.numpy as jnp
from jax.experimental import pallas as pl


def kernel(x, edge_index, W1, b1, W2, b2, W3, b3, W4, b4, gamma1, beta1, gamma2, beta2, gamma3, beta3):
    raise NotImplementedError("write your pallas kernel here")


# speedup vs baseline: 19.4956x; 19.4956x over previous
import jax
import jax.numpy as jnp
from jax import lax
from jax.experimental import pallas as pl
from jax.experimental.pallas import tpu as pltpu
from jax.experimental.pallas import tpu_sc as plsc

N = 10000
D = 128
E = 320000
NCORE = 2
NSUB = 16
NTILE = NCORE * NSUB
CHUNK = 128
NCH = 80
SLAB = 40
EPT = NCH * CHUNK
EPAD = EPT * NTILE
NPAD = 10240
RPT = NPAD // NSUB
EPS = 1e-5
BLK = 400
GRID = N // BLK


def _deg_body(dsts_hbm, zrow_hbm, ones_hbm, out_hbm,
              dstv, onesv, deg_sh, sem0, sem1, sem2, sem3):
    cid = lax.axis_index("c")
    sid = lax.axis_index("s")
    wid = cid * NSUB + sid
    pltpu.sync_copy(dsts_hbm.at[wid], dstv)
    pltpu.sync_copy(ones_hbm, onesv)
    pltpu.sync_copy(zrow_hbm, deg_sh.at[pl.ds(sid * RPT, RPT)])
    plsc.subcore_barrier()

    sems = (sem0, sem1, sem2, sem3)

    def quad(i, carry):
        j = 4 * i
        for k in range(4):
            pltpu.async_copy(onesv, deg_sh.at[dstv.at[j + k]], sems[k],
                             add=True)
        for k in range(4):
            pltpu.make_async_copy(onesv, deg_sh.at[dstv.at[j + k]],
                                  sems[k]).wait()
        return carry

    lax.fori_loop(0, NCH // 4, quad, 0)
    plsc.subcore_barrier()
    pltpu.sync_copy(deg_sh.at[pl.ds(sid * RPT, RPT)],
                    out_hbm.at[cid, pl.ds(sid * RPT, RPT)])


def _agg_body(y_hbm, srcs_hbm, dsts_hbm, zrows_hbm, out_hbm,
              srcv, dstv, bufa, bufb, acc_sh,
              gsema, gsemb, ssema, ssemb):
    cid = lax.axis_index("c")
    sid = lax.axis_index("s")
    wid = cid * NSUB + sid
    pltpu.sync_copy(zrows_hbm, acc_sh.at[pl.ds(sid * RPT, RPT)])
    plsc.subcore_barrier()

    for h in range(2):
        if h == 1:
            pltpu.make_async_copy(bufa, acc_sh.at[dstv.at[SLAB - 2]],
                                  ssema).wait()
            pltpu.make_async_copy(bufb, acc_sh.at[dstv.at[SLAB - 1]],
                                  ssemb).wait()
        pltpu.sync_copy(srcs_hbm.at[wid, pl.ds(h * SLAB, SLAB)], srcv)
        pltpu.sync_copy(dsts_hbm.at[wid, pl.ds(h * SLAB, SLAB)], dstv)
        pltpu.async_copy(y_hbm.at[srcv.at[0]], bufa, gsema).wait()
        pltpu.async_copy(bufa, acc_sh.at[dstv.at[0]], ssema, add=True)
        pltpu.async_copy(y_hbm.at[srcv.at[1]], bufb, gsemb).wait()
        pltpu.async_copy(bufb, acc_sh.at[dstv.at[1]], ssemb, add=True)

        def pair(i, carry):
            j0 = 2 * i + 2
            j1 = j0 + 1
            pltpu.make_async_copy(bufa, acc_sh.at[dstv.at[j0]], ssema).wait()
            pltpu.async_copy(y_hbm.at[srcv.at[j0]], bufa, gsema).wait()
            pltpu.async_copy(bufa, acc_sh.at[dstv.at[j0]], ssema, add=True)
            pltpu.make_async_copy(bufb, acc_sh.at[dstv.at[j1]], ssemb).wait()
            pltpu.async_copy(y_hbm.at[srcv.at[j1]], bufb, gsemb).wait()
            pltpu.async_copy(bufb, acc_sh.at[dstv.at[j1]], ssemb, add=True)
            return carry

        lax.fori_loop(0, SLAB // 2 - 1, pair, 0)

    pltpu.make_async_copy(bufa, acc_sh.at[dstv.at[SLAB - 2]], ssema).wait()
    pltpu.make_async_copy(bufb, acc_sh.at[dstv.at[SLAB - 1]], ssemb).wait()
    plsc.subcore_barrier()
    pltpu.sync_copy(acc_sh.at[pl.ds(sid * RPT, RPT)],
                    out_hbm.at[cid, pl.ds(sid * RPT, RPT)])


def _sc_mesh():
    return plsc.VectorSubcoreMesh(core_axis_name="c", subcore_axis_name="s")


def _deg(dsts, zrow, ones):
    fn = pl.kernel(
        _deg_body,
        out_type=jax.ShapeDtypeStruct((NCORE, NPAD), jnp.float32),
        mesh=_sc_mesh(),
        scratch_types=[
            pltpu.VMEM((NCH, CHUNK), jnp.int32),
            pltpu.VMEM((CHUNK,), jnp.float32),
            pltpu.VMEM_SHARED((NPAD,), jnp.float32),
            pltpu.SemaphoreType.DMA,
            pltpu.SemaphoreType.DMA,
            pltpu.SemaphoreType.DMA,
            pltpu.SemaphoreType.DMA,
        ],
    )
    return fn(dsts, zrow, ones)


def _agg(y, srcs, dsts, zrows):
    fn = pl.kernel(
        _agg_body,
        out_type=jax.ShapeDtypeStruct((NCORE, NPAD, D), jnp.float32),
        mesh=_sc_mesh(),
        scratch_types=[
            pltpu.VMEM((SLAB, CHUNK), jnp.int32),
            pltpu.VMEM((SLAB, CHUNK), jnp.int32),
            pltpu.VMEM((CHUNK, D), jnp.float32),
            pltpu.VMEM((CHUNK, D), jnp.float32),
            pltpu.VMEM_SHARED((NPAD, D), jnp.float32),
            pltpu.SemaphoreType.DMA,
            pltpu.SemaphoreType.DMA,
            pltpu.SemaphoreType.DMA,
            pltpu.SemaphoreType.DMA,
        ],
    )
    return fn(y, srcs, dsts, zrows)


def _pre_body(x_ref, w_ref, degp_ref, y_ref, dinv_ref):
    deg = 1.0 + degp_ref[0, :, 0] + degp_ref[1, :, 0]
    dinv = lax.rsqrt(deg)
    xw = jnp.dot(x_ref[...], w_ref[...], preferred_element_type=jnp.float32)
    y_ref[...] = xw * dinv[:, None]
    dinv_ref[...] = dinv[:, None]


def _pre(x, W1, degp):
    return pl.pallas_call(
        _pre_body,
        grid=(GRID,),
        in_specs=[
            pl.BlockSpec((BLK, D), lambda i: (i, 0)),
            pl.BlockSpec((D, D), lambda i: (0, 0)),
            pl.BlockSpec((NCORE, BLK, 1), lambda i: (0, i, 0)),
        ],
        out_specs=[
            pl.BlockSpec((BLK, D), lambda i: (i, 0)),
            pl.BlockSpec((BLK, 1), lambda i: (i, 0)),
        ],
        out_shape=[
            jax.ShapeDtypeStruct((N, D), jnp.float32),
            jax.ShapeDtypeStruct((N, 1), jnp.float32),
        ],
    )(x, W1, degp)


def _post_body(a_ref, y_ref, dinv_ref, b_ref, z_ref, s_ref, s2_ref):
    i = pl.program_id(0)
    agg = a_ref[0] + a_ref[1] + y_ref[...]
    z = jnp.maximum(agg * dinv_ref[...] + b_ref[...], 0.0)
    z_ref[...] = z

    @pl.when(i == 0)
    def _():
        s_ref[...] = jnp.zeros_like(s_ref)
        s2_ref[...] = jnp.zeros_like(s2_ref)

    s_ref[...] += jnp.sum(z, axis=0, keepdims=True)
    s2_ref[...] += jnp.sum(z * z, axis=0, keepdims=True)


def _post4_body(a_ref, y_ref, dinv_ref, b_ref, z_ref):
    agg = a_ref[0] + a_ref[1] + y_ref[...]
    z_ref[...] = jnp.maximum(agg * dinv_ref[...] + b_ref[...], 0.0)


def _post(agg, y, dinv, b, stats):
    in_specs = [
        pl.BlockSpec((NCORE, BLK, D), lambda i: (0, i, 0)),
        pl.BlockSpec((BLK, D), lambda i: (i, 0)),
        pl.BlockSpec((BLK, 1), lambda i: (i, 0)),
        pl.BlockSpec((1, D), lambda i: (0, 0)),
    ]
    if stats:
        return pl.pallas_call(
            _post_body,
            grid=(GRID,),
            in_specs=in_specs,
            out_specs=[
                pl.BlockSpec((BLK, D), lambda i: (i, 0)),
                pl.BlockSpec((1, D), lambda i: (0, 0)),
                pl.BlockSpec((1, D), lambda i: (0, 0)),
            ],
            out_shape=[
                jax.ShapeDtypeStruct((N, D), jnp.float32),
                jax.ShapeDtypeStruct((1, D), jnp.float32),
                jax.ShapeDtypeStruct((1, D), jnp.float32),
            ],
        )(agg, y, dinv, b)
    return pl.pallas_call(
        _post4_body,
        grid=(GRID,),
        in_specs=in_specs,
        out_specs=pl.BlockSpec((BLK, D), lambda i: (i, 0)),
        out_shape=jax.ShapeDtypeStruct((N, D), jnp.float32),
    )(agg, y, dinv, b)


def _bnmm_body(z_ref, s_ref, s2_ref, g_ref, be_ref, w_ref, dinv_ref, y_ref):
    m = s_ref[...] * (1.0 / N)
    v = s2_ref[...] * (1.0 / N) - m * m
    scale = g_ref[...] * lax.rsqrt(v + EPS)
    h = (z_ref[...] - m) * scale + be_ref[...]
    y = jnp.dot(h, w_ref[...], preferred_element_type=jnp.float32)
    y_ref[...] = y * dinv_ref[...]


def _bnmm(z, s, s2, gamma, beta, W, dinv):
    return pl.pallas_call(
        _bnmm_body,
        grid=(GRID,),
        in_specs=[
            pl.BlockSpec((BLK, D), lambda i: (i, 0)),
            pl.BlockSpec((1, D), lambda i: (0, 0)),
            pl.BlockSpec((1, D), lambda i: (0, 0)),
            pl.BlockSpec((1, D), lambda i: (0, 0)),
            pl.BlockSpec((1, D), lambda i: (0, 0)),
            pl.BlockSpec((D, D), lambda i: (0, 0)),
            pl.BlockSpec((BLK, 1), lambda i: (i, 0)),
        ],
        out_specs=pl.BlockSpec((BLK, D), lambda i: (i, 0)),
        out_shape=jax.ShapeDtypeStruct((N, D), jnp.float32),
    )(z, s, s2, gamma, beta, W, dinv)


def kernel(x, edge_index, W1, b1, W2, b2, W3, b3, W4, b4,
           gamma1, beta1, gamma2, beta2, gamma3, beta3):
    src = edge_index[0]
    dst = edge_index[1]
    pad = EPAD - E
    ar = jnp.arange(pad, dtype=jnp.int32)
    srcs = jnp.concatenate([src, (ar * 997) % N]).reshape(NTILE, NCH, CHUNK)
    dsts = jnp.concatenate([dst, N + ar % (NPAD - N)]).reshape(NTILE, NCH, CHUNK)
    zrows = jnp.zeros((RPT, D), jnp.float32)
    zrow = jnp.zeros((RPT,), jnp.float32)
    ones = jnp.ones((CHUNK,), jnp.float32)

    degp = _deg(dsts, zrow, ones).reshape(NCORE, NPAD, 1)
    y, dinv = _pre(x, W1, degp)

    bs = (b1.reshape(1, D), b2.reshape(1, D), b3.reshape(1, D), b4.reshape(1, D))
    Ws = (W2, W3, W4)
    gammas = (gamma1.reshape(1, D), gamma2.reshape(1, D), gamma3.reshape(1, D))
    betas = (beta1.reshape(1, D), beta2.reshape(1, D), beta3.reshape(1, D))

    for l in range(3):
        agg = _agg(y, srcs, dsts, zrows)
        z, s, s2 = _post(agg, y, dinv, bs[l], stats=True)
        y = _bnmm(z, s, s2, gammas[l], betas[l], Ws[l], dinv)
    agg = _agg(y, srcs, dsts, zrows)
    return _post(agg, y, dinv, bs[3], stats=False)


# speedup vs baseline: 19.7618x; 1.0137x over previous
import jax
import jax.numpy as jnp
from jax import lax
from jax.experimental import pallas as pl
from jax.experimental.pallas import tpu as pltpu
from jax.experimental.pallas import tpu_sc as plsc

N = 10000
D = 128
E = 320000
NCORE = 2
NSUB = 16
NTILE = NCORE * NSUB
CHUNK = 128
NCH = 80
SLAB = 40
EPT = NCH * CHUNK
EPAD = EPT * NTILE
NPAD = 10240
RPT = NPAD // NSUB
EPS = 1e-5
BLK = 400
GRID = N // BLK


def _deg_body(dsts_hbm, zrow_hbm, ones_hbm, out_hbm,
              dstv, onesv, deg_sh, sem0, sem1, sem2, sem3):
    cid = lax.axis_index("c")
    sid = lax.axis_index("s")
    wid = cid * NSUB + sid
    pltpu.sync_copy(dsts_hbm.at[wid], dstv)
    pltpu.sync_copy(ones_hbm, onesv)
    pltpu.sync_copy(zrow_hbm, deg_sh.at[pl.ds(sid * RPT, RPT)])
    plsc.subcore_barrier()

    sems = (sem0, sem1, sem2, sem3)

    def quad(i, carry):
        j = 4 * i
        for k in range(4):
            pltpu.async_copy(onesv, deg_sh.at[dstv.at[j + k]], sems[k],
                             add=True)
        for k in range(4):
            pltpu.make_async_copy(onesv, deg_sh.at[dstv.at[j + k]],
                                  sems[k]).wait()
        return carry

    lax.fori_loop(0, NCH // 4, quad, 0)
    plsc.subcore_barrier()
    pltpu.sync_copy(deg_sh.at[pl.ds(sid * RPT, RPT)],
                    out_hbm.at[cid, pl.ds(sid * RPT, RPT)])


def _agg_body(y_hbm, srcs_hbm, dsts_hbm, zrows_hbm, out_hbm,
              srcv, dstv, bufa, bufb, acc_sh,
              gsema, gsemb, ssema, ssemb):
    cid = lax.axis_index("c")
    sid = lax.axis_index("s")
    wid = cid * NSUB + sid
    pltpu.sync_copy(zrows_hbm, acc_sh.at[pl.ds(sid * RPT, RPT)])
    plsc.subcore_barrier()

    for h in range(2):
        if h == 1:
            pltpu.make_async_copy(bufa, acc_sh.at[dstv.at[SLAB - 2]],
                                  ssema).wait()
            pltpu.make_async_copy(bufb, acc_sh.at[dstv.at[SLAB - 1]],
                                  ssemb).wait()
        pltpu.sync_copy(srcs_hbm.at[wid, pl.ds(h * SLAB, SLAB)], srcv)
        pltpu.sync_copy(dsts_hbm.at[wid, pl.ds(h * SLAB, SLAB)], dstv)
        pltpu.async_copy(y_hbm.at[srcv.at[0]], bufa, gsema).wait()
        pltpu.async_copy(bufa, acc_sh.at[dstv.at[0]], ssema, add=True)
        pltpu.async_copy(y_hbm.at[srcv.at[1]], bufb, gsemb).wait()
        pltpu.async_copy(bufb, acc_sh.at[dstv.at[1]], ssemb, add=True)

        def pair(i, carry):
            j0 = 2 * i + 2
            j1 = j0 + 1
            pltpu.make_async_copy(bufa, acc_sh.at[dstv.at[j0]], ssema).wait()
            pltpu.async_copy(y_hbm.at[srcv.at[j0]], bufa, gsema).wait()
            pltpu.async_copy(bufa, acc_sh.at[dstv.at[j0]], ssema, add=True)
            pltpu.make_async_copy(bufb, acc_sh.at[dstv.at[j1]], ssemb).wait()
            pltpu.async_copy(y_hbm.at[srcv.at[j1]], bufb, gsemb).wait()
            pltpu.async_copy(bufb, acc_sh.at[dstv.at[j1]], ssemb, add=True)
            return carry

        lax.fori_loop(0, SLAB // 2 - 1, pair, 0)

    pltpu.make_async_copy(bufa, acc_sh.at[dstv.at[SLAB - 2]], ssema).wait()
    pltpu.make_async_copy(bufb, acc_sh.at[dstv.at[SLAB - 1]], ssemb).wait()
    plsc.subcore_barrier()
    pltpu.sync_copy(acc_sh.at[pl.ds(sid * RPT, RPT)],
                    out_hbm.at[cid, pl.ds(sid * RPT, RPT)])


def _sc_mesh():
    return plsc.VectorSubcoreMesh(core_axis_name="c", subcore_axis_name="s")


def _deg(dsts, zrow, ones):
    fn = pl.kernel(
        _deg_body,
        out_type=jax.ShapeDtypeStruct((NCORE, NPAD), jnp.float32),
        mesh=_sc_mesh(),
        scratch_types=[
            pltpu.VMEM((NCH, CHUNK), jnp.int32),
            pltpu.VMEM((CHUNK,), jnp.float32),
            pltpu.VMEM_SHARED((NPAD,), jnp.float32),
            pltpu.SemaphoreType.DMA,
            pltpu.SemaphoreType.DMA,
            pltpu.SemaphoreType.DMA,
            pltpu.SemaphoreType.DMA,
        ],
    )
    return fn(dsts, zrow, ones)


def _agg(y, srcs, dsts, zrows):
    fn = pl.kernel(
        _agg_body,
        out_type=jax.ShapeDtypeStruct((NCORE, NPAD, D), jnp.float32),
        mesh=_sc_mesh(),
        scratch_types=[
            pltpu.VMEM((SLAB, CHUNK), jnp.int32),
            pltpu.VMEM((SLAB, CHUNK), jnp.int32),
            pltpu.VMEM((CHUNK, D), jnp.float32),
            pltpu.VMEM((CHUNK, D), jnp.float32),
            pltpu.VMEM_SHARED((NPAD, D), jnp.float32),
            pltpu.SemaphoreType.DMA,
            pltpu.SemaphoreType.DMA,
            pltpu.SemaphoreType.DMA,
            pltpu.SemaphoreType.DMA,
        ],
    )
    return fn(y, srcs, dsts, zrows)


def _pre_body(x_ref, w_ref, degp_ref, y_ref, dinv_ref):
    deg = 1.0 + degp_ref[0, :, 0] + degp_ref[1, :, 0]
    dinv = lax.rsqrt(deg)
    xw = jnp.dot(x_ref[...], w_ref[...], preferred_element_type=jnp.float32)
    y_ref[...] = xw * dinv[:, None]
    dinv_ref[...] = dinv[:, None]


def _pre(x, W1, degp):
    return pl.pallas_call(
        _pre_body,
        grid=(GRID,),
        in_specs=[
            pl.BlockSpec((BLK, D), lambda i: (i, 0)),
            pl.BlockSpec((D, D), lambda i: (0, 0)),
            pl.BlockSpec((NCORE, BLK, 1), lambda i: (0, i, 0)),
        ],
        out_specs=[
            pl.BlockSpec((BLK, D), lambda i: (i, 0)),
            pl.BlockSpec((BLK, 1), lambda i: (i, 0)),
        ],
        out_shape=[
            jax.ShapeDtypeStruct((N, D), jnp.float32),
            jax.ShapeDtypeStruct((N, 1), jnp.float32),
        ],
    )(x, W1, degp)


def _post4_body(a_ref, y_ref, dinv_ref, b_ref, z_ref):
    agg = a_ref[0] + a_ref[1] + y_ref[...]
    z_ref[...] = jnp.maximum(agg * dinv_ref[...] + b_ref[...], 0.0)


def _post4(agg, y, dinv, b):
    return pl.pallas_call(
        _post4_body,
        grid=(GRID,),
        in_specs=[
            pl.BlockSpec((NCORE, BLK, D), lambda i: (0, i, 0)),
            pl.BlockSpec((BLK, D), lambda i: (i, 0)),
            pl.BlockSpec((BLK, 1), lambda i: (i, 0)),
            pl.BlockSpec((1, D), lambda i: (0, 0)),
        ],
        out_specs=pl.BlockSpec((BLK, D), lambda i: (i, 0)),
        out_shape=jax.ShapeDtypeStruct((N, D), jnp.float32),
    )(agg, y, dinv, b)


def _postbn_body(a_ref, y_ref, dinv_ref, b_ref, g_ref, be_ref, w_ref,
                 out_ref, z_scr, s_scr, s2_scr):
    i = pl.program_id(0)

    @pl.when(i < GRID)
    def _phase0():
        agg = a_ref[0] + a_ref[1] + y_ref[...]
        z = jnp.maximum(agg * dinv_ref[...] + b_ref[...], 0.0)
        z_scr[pl.ds(i * BLK, BLK), :] = z

        @pl.when(i == 0)
        def _():
            s_scr[...] = jnp.zeros_like(s_scr)
            s2_scr[...] = jnp.zeros_like(s2_scr)

        s_scr[...] += jnp.sum(z, axis=0, keepdims=True)
        s2_scr[...] += jnp.sum(z * z, axis=0, keepdims=True)

    @pl.when(i >= GRID)
    def _phase1():
        k = i - GRID
        z = z_scr[pl.ds(k * BLK, BLK), :]
        m = s_scr[...] * (1.0 / N)
        v = s2_scr[...] * (1.0 / N) - m * m
        scale = g_ref[...] * lax.rsqrt(v + EPS)
        h = (z - m) * scale + be_ref[...]
        y = jnp.dot(h, w_ref[...], preferred_element_type=jnp.float32)
        out_ref[...] = y * dinv_ref[...]


def _postbn(agg, y, dinv, b, gamma, beta, W):
    return pl.pallas_call(
        _postbn_body,
        grid=(2 * GRID,),
        in_specs=[
            pl.BlockSpec((NCORE, BLK, D),
                         lambda i: (0, jnp.where(i < GRID, i, 0), 0)),
            pl.BlockSpec((BLK, D), lambda i: (jnp.where(i < GRID, i, 0), 0)),
            pl.BlockSpec((BLK, 1),
                         lambda i: (jnp.where(i < GRID, i, i - GRID), 0)),
            pl.BlockSpec((1, D), lambda i: (0, 0)),
            pl.BlockSpec((1, D), lambda i: (0, 0)),
            pl.BlockSpec((1, D), lambda i: (0, 0)),
            pl.BlockSpec((D, D), lambda i: (0, 0)),
        ],
        out_specs=pl.BlockSpec((BLK, D),
                               lambda i: (jnp.where(i < GRID, 0, i - GRID), 0)),
        out_shape=jax.ShapeDtypeStruct((N, D), jnp.float32),
        scratch_shapes=[
            pltpu.VMEM((N, D), jnp.float32),
            pltpu.VMEM((1, D), jnp.float32),
            pltpu.VMEM((1, D), jnp.float32),
        ],
    )(agg, y, dinv, b, gamma, beta, W)


def kernel(x, edge_index, W1, b1, W2, b2, W3, b3, W4, b4,
           gamma1, beta1, gamma2, beta2, gamma3, beta3):
    src = edge_index[0]
    dst = edge_index[1]
    pad = EPAD - E
    ar = jnp.arange(pad, dtype=jnp.int32)
    srcs = jnp.concatenate([src, (ar * 997) % N]).reshape(NTILE, NCH, CHUNK)
    dsts = jnp.concatenate([dst, N + ar % (NPAD - N)]).reshape(NTILE, NCH, CHUNK)
    zrows = jnp.zeros((RPT, D), jnp.float32)
    zrow = jnp.zeros((RPT,), jnp.float32)
    ones = jnp.ones((CHUNK,), jnp.float32)

    degp = _deg(dsts, zrow, ones).reshape(NCORE, NPAD, 1)
    y, dinv = _pre(x, W1, degp)

    bs = (b1.reshape(1, D), b2.reshape(1, D), b3.reshape(1, D), b4.reshape(1, D))
    Ws = (W2, W3, W4)
    gammas = (gamma1.reshape(1, D), gamma2.reshape(1, D), gamma3.reshape(1, D))
    betas = (beta1.reshape(1, D), beta2.reshape(1, D), beta3.reshape(1, D))

    for l in range(3):
        agg = _agg(y, srcs, dsts, zrows)
        y = _postbn(agg, y, dinv, bs[l], gammas[l], betas[l], Ws[l])
    agg = _agg(y, srcs, dsts, zrows)
    return _post4(agg, y, dinv, bs[3])


# speedup vs baseline: 20.7805x; 1.0515x over previous
import jax
import jax.numpy as jnp
from jax import lax
from jax.experimental import pallas as pl
from jax.experimental.pallas import tpu as pltpu
from jax.experimental.pallas import tpu_sc as plsc

N = 10000
D = 128
E = 320000
NCORE = 2
NSUB = 16
NTILE = NCORE * NSUB
CHUNK = 128
NCH = 80
ACH = 64
ANCH = 160
ASLAB = 40
EPT = NCH * CHUNK
EPAD = EPT * NTILE
NPAD = 10240
RPT = NPAD // NSUB
EPS = 1e-5
BLK = 400
GRID = N // BLK


def _deg_body(dsts_hbm, zrow_hbm, ones_hbm, out_hbm,
              dstv, onesv, deg_sh, sem0, sem1, sem2, sem3):
    cid = lax.axis_index("c")
    sid = lax.axis_index("s")
    wid = cid * NSUB + sid
    pltpu.sync_copy(dsts_hbm.at[wid], dstv)
    pltpu.sync_copy(ones_hbm, onesv)
    pltpu.sync_copy(zrow_hbm, deg_sh.at[pl.ds(sid * RPT, RPT)])
    plsc.subcore_barrier()

    sems = (sem0, sem1, sem2, sem3)

    def quad(i, carry):
        j = 4 * i
        for k in range(4):
            pltpu.async_copy(onesv, deg_sh.at[dstv.at[j + k]], sems[k],
                             add=True)
        for k in range(4):
            pltpu.make_async_copy(onesv, deg_sh.at[dstv.at[j + k]],
                                  sems[k]).wait()
        return carry

    lax.fori_loop(0, NCH // 4, quad, 0)
    plsc.subcore_barrier()
    pltpu.sync_copy(deg_sh.at[pl.ds(sid * RPT, RPT)],
                    out_hbm.at[cid, pl.ds(sid * RPT, RPT)])


def _agg_body(y_hbm, srcs_hbm, dsts_hbm, zrows_hbm, out_hbm,
              srcv, dstv, buf0, buf1, buf2, buf3, acc_sh,
              gsem0, gsem1, gsem2, gsem3, ssem0, ssem1, ssem2, ssem3):
    cid = lax.axis_index("c")
    sid = lax.axis_index("s")
    wid = cid * NSUB + sid
    bufs = (buf0, buf1, buf2, buf3)
    gsems = (gsem0, gsem1, gsem2, gsem3)
    ssems = (ssem0, ssem1, ssem2, ssem3)
    pltpu.sync_copy(zrows_hbm, acc_sh.at[pl.ds(sid * RPT, RPT)])
    plsc.subcore_barrier()

    for h in range(ANCH // ASLAB):
        if h > 0:
            for k in range(4):
                pltpu.make_async_copy(bufs[k], acc_sh.at[dstv.at[k]],
                                      ssems[k]).wait()
        pltpu.sync_copy(srcs_hbm.at[wid, pl.ds(h * ASLAB, ASLAB)], srcv)
        pltpu.sync_copy(dsts_hbm.at[wid, pl.ds(h * ASLAB, ASLAB)], dstv)
        for k in range(4):
            pltpu.async_copy(y_hbm.at[srcv.at[k]], bufs[k], gsems[k])
        for k in range(4):
            pltpu.make_async_copy(y_hbm.at[srcv.at[k]], bufs[k],
                                  gsems[k]).wait()
            pltpu.async_copy(bufs[k], acc_sh.at[dstv.at[k]], ssems[k],
                             add=True)

        def quad(i, carry):
            j = 4 * i + 4
            for k in range(4):
                pltpu.make_async_copy(bufs[k], acc_sh.at[dstv.at[j + k]],
                                      ssems[k]).wait()
                pltpu.async_copy(y_hbm.at[srcv.at[j + k]], bufs[k], gsems[k])
            for k in range(4):
                pltpu.make_async_copy(y_hbm.at[srcv.at[j + k]], bufs[k],
                                      gsems[k]).wait()
                pltpu.async_copy(bufs[k], acc_sh.at[dstv.at[j + k]],
                                 ssems[k], add=True)
            return carry

        lax.fori_loop(0, ASLAB // 4 - 1, quad, 0)

    for k in range(4):
        pltpu.make_async_copy(bufs[k], acc_sh.at[dstv.at[k]], ssems[k]).wait()
    plsc.subcore_barrier()
    pltpu.sync_copy(acc_sh.at[pl.ds(sid * RPT, RPT)],
                    out_hbm.at[cid, pl.ds(sid * RPT, RPT)])


def _sc_mesh():
    return plsc.VectorSubcoreMesh(core_axis_name="c", subcore_axis_name="s")


def _deg(dsts, zrow, ones):
    fn = pl.kernel(
        _deg_body,
        out_type=jax.ShapeDtypeStruct((NCORE, NPAD), jnp.float32),
        mesh=_sc_mesh(),
        scratch_types=[
            pltpu.VMEM((NCH, CHUNK), jnp.int32),
            pltpu.VMEM((CHUNK,), jnp.float32),
            pltpu.VMEM_SHARED((NPAD,), jnp.float32),
            pltpu.SemaphoreType.DMA,
            pltpu.SemaphoreType.DMA,
            pltpu.SemaphoreType.DMA,
            pltpu.SemaphoreType.DMA,
        ],
    )
    return fn(dsts, zrow, ones)


def _agg(y, srcs, dsts, zrows):
    fn = pl.kernel(
        _agg_body,
        out_type=jax.ShapeDtypeStruct((NCORE, NPAD, D), jnp.float32),
        mesh=_sc_mesh(),
        scratch_types=[
            pltpu.VMEM((ASLAB, ACH), jnp.int32),
            pltpu.VMEM((ASLAB, ACH), jnp.int32),
            pltpu.VMEM((ACH, D), jnp.float32),
            pltpu.VMEM((ACH, D), jnp.float32),
            pltpu.VMEM((ACH, D), jnp.float32),
            pltpu.VMEM((ACH, D), jnp.float32),
            pltpu.VMEM_SHARED((NPAD, D), jnp.float32),
            pltpu.SemaphoreType.DMA,
            pltpu.SemaphoreType.DMA,
            pltpu.SemaphoreType.DMA,
            pltpu.SemaphoreType.DMA,
            pltpu.SemaphoreType.DMA,
            pltpu.SemaphoreType.DMA,
            pltpu.SemaphoreType.DMA,
            pltpu.SemaphoreType.DMA,
        ],
    )
    return fn(y, srcs, dsts, zrows)


def _pre_body(x_ref, w_ref, degp_ref, y_ref, dinv_ref):
    deg = 1.0 + degp_ref[0, :, 0] + degp_ref[1, :, 0]
    dinv = lax.rsqrt(deg)
    xw = jnp.dot(x_ref[...], w_ref[...], preferred_element_type=jnp.float32)
    y_ref[...] = xw * dinv[:, None]
    dinv_ref[...] = dinv[:, None]


def _pre(x, W1, degp):
    return pl.pallas_call(
        _pre_body,
        grid=(GRID,),
        in_specs=[
            pl.BlockSpec((BLK, D), lambda i: (i, 0)),
            pl.BlockSpec((D, D), lambda i: (0, 0)),
            pl.BlockSpec((NCORE, BLK, 1), lambda i: (0, i, 0)),
        ],
        out_specs=[
            pl.BlockSpec((BLK, D), lambda i: (i, 0)),
            pl.BlockSpec((BLK, 1), lambda i: (i, 0)),
        ],
        out_shape=[
            jax.ShapeDtypeStruct((N, D), jnp.float32),
            jax.ShapeDtypeStruct((N, 1), jnp.float32),
        ],
    )(x, W1, degp)


def _post4_body(a_ref, y_ref, dinv_ref, b_ref, z_ref):
    agg = a_ref[0] + a_ref[1] + y_ref[...]
    z_ref[...] = jnp.maximum(agg * dinv_ref[...] + b_ref[...], 0.0)


def _post4(agg, y, dinv, b):
    return pl.pallas_call(
        _post4_body,
        grid=(GRID,),
        in_specs=[
            pl.BlockSpec((NCORE, BLK, D), lambda i: (0, i, 0)),
            pl.BlockSpec((BLK, D), lambda i: (i, 0)),
            pl.BlockSpec((BLK, 1), lambda i: (i, 0)),
            pl.BlockSpec((1, D), lambda i: (0, 0)),
        ],
        out_specs=pl.BlockSpec((BLK, D), lambda i: (i, 0)),
        out_shape=jax.ShapeDtypeStruct((N, D), jnp.float32),
    )(agg, y, dinv, b)


def _postbn_body(a_ref, y_ref, dinv_ref, b_ref, g_ref, be_ref, w_ref,
                 out_ref, z_scr, s_scr, s2_scr):
    i = pl.program_id(0)

    @pl.when(i < GRID)
    def _phase0():
        agg = a_ref[0] + a_ref[1] + y_ref[...]
        z = jnp.maximum(agg * dinv_ref[...] + b_ref[...], 0.0)
        z_scr[pl.ds(i * BLK, BLK), :] = z

        @pl.when(i == 0)
        def _():
            s_scr[...] = jnp.zeros_like(s_scr)
            s2_scr[...] = jnp.zeros_like(s2_scr)

        s_scr[...] += jnp.sum(z, axis=0, keepdims=True)
        s2_scr[...] += jnp.sum(z * z, axis=0, keepdims=True)

    @pl.when(i >= GRID)
    def _phase1():
        k = i - GRID
        z = z_scr[pl.ds(k * BLK, BLK), :]
        m = s_scr[...] * (1.0 / N)
        v = s2_scr[...] * (1.0 / N) - m * m
        scale = g_ref[...] * lax.rsqrt(v + EPS)
        h = (z - m) * scale + be_ref[...]
        y = jnp.dot(h, w_ref[...], preferred_element_type=jnp.float32)
        out_ref[...] = y * dinv_ref[...]


def _postbn(agg, y, dinv, b, gamma, beta, W):
    return pl.pallas_call(
        _postbn_body,
        grid=(2 * GRID,),
        in_specs=[
            pl.BlockSpec((NCORE, BLK, D),
                         lambda i: (0, jnp.where(i < GRID, i, 0), 0)),
            pl.BlockSpec((BLK, D), lambda i: (jnp.where(i < GRID, i, 0), 0)),
            pl.BlockSpec((BLK, 1),
                         lambda i: (jnp.where(i < GRID, i, i - GRID), 0)),
            pl.BlockSpec((1, D), lambda i: (0, 0)),
            pl.BlockSpec((1, D), lambda i: (0, 0)),
            pl.BlockSpec((1, D), lambda i: (0, 0)),
            pl.BlockSpec((D, D), lambda i: (0, 0)),
        ],
        out_specs=pl.BlockSpec((BLK, D),
                               lambda i: (jnp.where(i < GRID, 0, i - GRID), 0)),
        out_shape=jax.ShapeDtypeStruct((N, D), jnp.float32),
        scratch_shapes=[
            pltpu.VMEM((N, D), jnp.float32),
            pltpu.VMEM((1, D), jnp.float32),
            pltpu.VMEM((1, D), jnp.float32),
        ],
    )(agg, y, dinv, b, gamma, beta, W)


def kernel(x, edge_index, W1, b1, W2, b2, W3, b3, W4, b4,
           gamma1, beta1, gamma2, beta2, gamma3, beta3):
    src = edge_index[0]
    dst = edge_index[1]
    pad = EPAD - E
    ar = jnp.arange(pad, dtype=jnp.int32)
    src_flat = jnp.concatenate([src, (ar * 997) % N])
    dst_flat = jnp.concatenate([dst, N + ar % (NPAD - N)])
    srcs = src_flat.reshape(NTILE, NCH, CHUNK)
    dsts = dst_flat.reshape(NTILE, NCH, CHUNK)
    srcsa = src_flat.reshape(NTILE, ANCH, ACH)
    dstsa = dst_flat.reshape(NTILE, ANCH, ACH)
    zrows = jnp.zeros((RPT, D), jnp.float32)
    zrow = jnp.zeros((RPT,), jnp.float32)
    ones = jnp.ones((CHUNK,), jnp.float32)

    degp = _deg(dsts, zrow, ones).reshape(NCORE, NPAD, 1)
    y, dinv = _pre(x, W1, degp)

    bs = (b1.reshape(1, D), b2.reshape(1, D), b3.reshape(1, D), b4.reshape(1, D))
    Ws = (W2, W3, W4)
    gammas = (gamma1.reshape(1, D), gamma2.reshape(1, D), gamma3.reshape(1, D))
    betas = (beta1.reshape(1, D), beta2.reshape(1, D), beta3.reshape(1, D))

    for l in range(3):
        agg = _agg(y, srcsa, dstsa, zrows)
        y = _postbn(agg, y, dinv, bs[l], gammas[l], betas[l], Ws[l])
    agg = _agg(y, srcsa, dstsa, zrows)
    return _post4(agg, y, dinv, bs[3])
